# baseline (speedup 1.0000x reference)
import functools

import jax
import jax.numpy as jnp
from jax import lax
from jax.experimental import pallas as pl
from jax.experimental.pallas import tpu as pltpu
from jax.experimental.pallas import tpu_sc as plsc

V = 8192
N = 1024
RAY_SAMPLES = 8
INF_BITS = 0x7F800000
RADIX = 2048
RMASK = RADIX - 1
NW = 32
RPW = N // NW
PAD = 16
EXIT_LOG_T = -9.2104

_TC_BLOCK = 32


def _tc_slab_kernel(vox_ref, rays_ref, rd_ref, key_ref, x_ref, cnt_ref, skb_ref):
    vox = vox_ref[...]
    bminx, bminy, bminz = vox[0:1], vox[1:2], vox[2:3]
    bmaxx, bmaxy, bmaxz = vox[3:4], vox[4:5], vox[5:6]
    density, morton_f = vox[6:7], vox[7:8]
    px, py, pz = vox[8:9], vox[9:10], vox[10:11]

    rays = rays_ref[...]
    ox, oy, oz = rays[:, 0:1], rays[:, 1:2], rays[:, 2:3]
    ivx, ivy, ivz = rays[:, 3:4], rays[:, 4:5], rays[:, 5:6]

    t1 = (bminx - ox) * ivx
    t2 = (bmaxx - ox) * ivx
    tn = jnp.minimum(t1, t2)
    tf = jnp.maximum(t1, t2)
    t1 = (bminy - oy) * ivy
    t2 = (bmaxy - oy) * ivy
    tn = jnp.maximum(tn, jnp.minimum(t1, t2))
    tf = jnp.minimum(tf, jnp.maximum(t1, t2))
    t1 = (bminz - oz) * ivz
    t2 = (bmaxz - oz) * ivz
    tn = jnp.maximum(tn, jnp.minimum(t1, t2))
    tf = jnp.minimum(tf, jnp.maximum(t1, t2))

    valid = (tn <= tf) & (tf > 0.0)
    tnc = jnp.maximum(tn, 0.0)
    kb = lax.bitcast_convert_type(tnc, jnp.int32)
    kb = jnp.where(tnc == 0.0, 0, kb)
    kb = jnp.where(valid, kb, INF_BITS)
    key_ref[...] = kb
    x_ref[...] = density * ((tf - tnc) * (1.0 / RAY_SAMPLES))
    nblk = valid.shape[0]
    v16 = valid.astype(jnp.int32).reshape(nblk, V // 16, 16)
    cnt_ref[...] = jnp.sum(v16, axis=2)

    rd = rd_ref[...]
    mean = jnp.mean(rd[:, 0:3], axis=0, keepdims=True)
    dots = px * mean[:, 0:1] + py * mean[:, 1:2] + pz * mean[:, 2:3]
    skf = morton_f + dots * 1e-06
    bu = lax.bitcast_convert_type(skf, jnp.uint32)
    mapped = jnp.where(bu >> 31 != 0, ~bu, bu | jnp.uint32(0x80000000))
    skb = lax.bitcast_convert_type(mapped, jnp.int32)
    skb_ref[...] = jnp.broadcast_to(skb, (8, V))


def _radix_pass(c, src_k, src_i, dst_k, dst_i, hist, shift, lane, radix):
    rmask = radix - 1
    nv = (c + 15) // 16

    def clr(m, carry):
        hist[pl.ds(m * 16, 16)] = jnp.zeros((16,), jnp.int32)
        return carry
    lax.fori_loop(0, radix // 16, clr, 0)

    def hst(j, carry):
        k = src_k[pl.ds(j * 16, 16)]
        d = lax.shift_right_logical(k, shift) & rmask
        lm = (j * 16 + lane) < c
        cnt, last = plsc.scan_count(d, mask=lm)
        plsc.addupdate_scatter(hist, [d], cnt, mask=last)
        return carry
    lax.fori_loop(0, nv, hst, 0)

    def scn(m, carry):
        h = hist[pl.ds(m * 16, 16)]
        incl = plsc.cumsum(h)
        hist[pl.ds(m * 16, 16)] = incl - h + carry
        return carry + jnp.sum(h)
    lax.fori_loop(0, radix // 16, scn, jnp.int32(0))

    def prm(j, carry):
        k = src_k[pl.ds(j * 16, 16)]
        pidx = src_i[pl.ds(j * 16, 16)]
        lm = (j * 16 + lane) < c
        d = lax.shift_right_logical(k, shift) & rmask
        cnt, last = plsc.scan_count(d, mask=lm)
        base = plsc.load_gather(hist, [d])
        pos = base + cnt - 1
        plsc.store_scatter(dst_k, [pos], k, mask=lm)
        plsc.store_scatter(dst_i, [pos], pidx, mask=lm)
        plsc.addupdate_scatter(hist, [d], cnt, mask=last)
        return carry
    lax.fori_loop(0, nv, prm, 0)


def _sc_render_kernel(key_hbm, x_hbm, cnt_hbm, skb_hbm, colr_hbm, colg_hbm,
                      colb_hbm, den_hbm, r_hbm, g_hbm, b_hbm, d_hbm, a_hbm,
                      rankv, krow, xrow, KA, IA, KB, IB,
                      colr_v, colg_v, colb_v, den_v, hist, cnts,
                      obuf_r, obuf_g, obuf_b, obuf_d, obuf_a):
    wid = lax.axis_index("s") * 2 + lax.axis_index("c")
    lane = jnp.arange(16, dtype=jnp.int32)

    pltpu.sync_copy(skb_hbm, KA.at[pl.ds(0, V)])

    def fill(j, carry):
        IA[pl.ds(j * 16, 16)] = j * 16 + lane
        return carry
    lax.fori_loop(0, V // 16, fill, 0)
    cV = jnp.int32(V)
    _radix_pass(cV, KA, IA, KB, IB, hist, 0, lane, RADIX)
    _radix_pass(cV, KB, IB, KA, IA, hist, 11, lane, RADIX)
    _radix_pass(cV, KA, IA, KB, IB, hist, 22, lane, RADIX)

    def inv(j, carry):
        ov = IB[pl.ds(j * 16, 16)]
        plsc.store_scatter(rankv, [ov], j * 16 + lane)
        return carry
    lax.fori_loop(0, V // 16, inv, 0)

    pltpu.sync_copy(colr_hbm, colr_v)
    pltpu.sync_copy(colg_hbm, colg_v)
    pltpu.sync_copy(colb_hbm, colb_v)
    pltpu.sync_copy(den_hbm, den_v)

    def ray_body(rr, carry):
        ray = wid * RPW + rr
        pltpu.sync_copy(key_hbm.at[ray], krow)
        pltpu.sync_copy(x_hbm.at[ray], xrow)
        pltpu.sync_copy(cnt_hbm.at[ray], cnts.at[pl.ds(0, V // 16)])

        def pfx(m, carry):
            h = cnts[pl.ds(m * 16, 16)]
            incl = plsc.cumsum(h)
            cnts[pl.ds(m * 16, 16)] = incl - h + carry
            return carry + jnp.sum(h)
        c = lax.fori_loop(0, V // 256, pfx, jnp.int32(0))

        def cmp_body(j, carry):
            k = krow[pl.ds(j * 16, 16)]
            m = k < INF_BITS
            off = cnts[pl.ds(j, 16)][0]
            plsc.store_compressed(KA.at[pl.ds(off, 16)], k, mask=m)
            plsc.store_compressed(IA.at[pl.ds(off, 16)], j * 16 + lane, mask=m)
            return carry
        lax.fori_loop(0, V // 16, cmp_body, 0)

        _radix_pass(c, KA, IA, KB, IB, hist, 0, lane, 256)
        _radix_pass(c, KB, IB, KA, IA, hist, 8, lane, 256)
        _radix_pass(c, KA, IA, KB, IB, hist, 16, lane, 256)
        _radix_pass(c, KB, IB, KA, IA, hist, 24, lane, 256)

        nv = (c + 15) // 16

        def zc(j, z):
            ks = KA[pl.ds(j * 16, 16)]
            zm = (ks == 0) & ((j * 16 + lane) < c)
            return z + jnp.sum(zm.astype(jnp.int32))
        z = lax.fori_loop(0, nv, zc, jnp.int32(0))
        zv = (z + 15) // 16

        def zload(j, carry):
            kidx = IA[pl.ds(j * 16, 16)]
            rv = plsc.load_gather(rankv, [kidx])
            KB[pl.ds(j * 16, 16)] = rv
            IB[pl.ds(j * 16, 16)] = kidx
            return carry
        lax.fori_loop(0, zv, zload, 0)
        _radix_pass(z, KB, IB, KA, IA, hist, 0, lane, 128)
        _radix_pass(z, KA, IA, KB, IB, hist, 7, lane, 128)

        def zstore(j, carry):
            pos = j * 16 + lane
            zm = pos < z
            plsc.store_scatter(IA, [pos], IB[pl.ds(j * 16, 16)], mask=zm)
            plsc.store_scatter(KA, [pos], jnp.zeros((16,), jnp.int32), mask=zm)
            return carry
        lax.fori_loop(0, zv, zstore, 0)
        zero16 = jnp.zeros((16,), jnp.float32)

        def cc(st):
            jj, sumL, sumLp, aR, aG, aB, aD = st
            return (jj < nv) & (sumL >= EXIT_LOG_T)

        def cb(st):
            jj, sumL, sumLp, aR, aG, aB, aD = st
            kidx = IA[pl.ds(jj * 16, 16)]
            ks = KA[pl.ds(jj * 16, 16)]
            lm = (jj * 16 + lane) < c
            xg = plsc.load_gather(xrow, [kidx])
            xg = jnp.where(lm, xg, 0.0)
            l = -xg
            incl = plsc.cumsum(l)
            S = sumL + (incl - l)
            T = jnp.exp(S)
            e = jnp.exp(l)
            o = 1.0 - e
            proc = (T >= 1e-4) & lm
            w = jnp.where(proc, T * o, 0.0)
            tnc = plsc.bitcast(ks, jnp.float32)
            dg = plsc.load_gather(den_v, [kidx])
            mg = tnc + 4.0 * xg / dg
            cr = plsc.load_gather(colr_v, [kidx])
            cg = plsc.load_gather(colg_v, [kidx])
            cb_ = plsc.load_gather(colb_v, [kidx])
            aR = aR + w * cr
            aG = aG + w * cg
            aB = aB + w * cb_
            aD = aD + w * mg
            sumLp = sumLp + jnp.sum(jnp.where(proc, l, 0.0))
            sumL = sumL + jnp.sum(l)
            return (jj + 1, sumL, sumLp, aR, aG, aB, aD)

        st0 = (jnp.int32(0), jnp.float32(0), jnp.float32(0),
               zero16, zero16, zero16, zero16)
        _, _, sumLp, aR, aG, aB, aD = lax.while_loop(cc, cb, st0)

        rrv = jnp.full((16,), rr, jnp.int32)
        one_lane = lane == 0
        plsc.store_scatter(obuf_r, [rrv], jnp.full((16,), jnp.sum(aR)), mask=one_lane)
        plsc.store_scatter(obuf_g, [rrv], jnp.full((16,), jnp.sum(aG)), mask=one_lane)
        plsc.store_scatter(obuf_b, [rrv], jnp.full((16,), jnp.sum(aB)), mask=one_lane)
        plsc.store_scatter(obuf_d, [rrv], jnp.full((16,), jnp.sum(aD)), mask=one_lane)
        alpha_v = 1.0 - jnp.exp(jnp.full((16,), sumLp))
        plsc.store_scatter(obuf_a, [rrv], alpha_v, mask=one_lane)
        return carry

    lax.fori_loop(0, RPW, ray_body, 0)

    base = wid * RPW
    pltpu.sync_copy(obuf_r, r_hbm.at[pl.ds(base, RPW)])
    pltpu.sync_copy(obuf_g, g_hbm.at[pl.ds(base, RPW)])
    pltpu.sync_copy(obuf_b, b_hbm.at[pl.ds(base, RPW)])
    pltpu.sync_copy(obuf_d, d_hbm.at[pl.ds(base, RPW)])
    pltpu.sync_copy(obuf_a, a_hbm.at[pl.ds(base, RPW)])


def kernel(positions, sizes, densities, colors, morton_codes, ray_origins,
           ray_directions):
    half = sizes * 0.5
    bmin = positions - half[:, None]
    bmax = positions + half[:, None]
    density = jnp.exp(densities)
    vox = jnp.zeros((16, V), jnp.float32)
    vox = vox.at[0:3].set(bmin.T)
    vox = vox.at[3:6].set(bmax.T)
    vox = vox.at[6].set(density)
    vox = vox.at[7].set(morton_codes.astype(jnp.float32))
    vox = vox.at[8:11].set(positions.T)

    inv_dir = 1.0 / (ray_directions + 1e-08)
    rays = jnp.zeros((N, 16), jnp.float32)
    rays = rays.at[:, 0:3].set(ray_origins)
    rays = rays.at[:, 3:6].set(inv_dir)
    rd_full = jnp.zeros((N, 8), jnp.float32)
    rd_full = rd_full.at[:, 0:3].set(ray_directions)

    color = jax.nn.sigmoid(colors[:, :3])
    colr = jnp.asarray(color[:, 0])
    colg = jnp.asarray(color[:, 1])
    colb = jnp.asarray(color[:, 2])

    grid = N // _TC_BLOCK
    keymat, xmat, cntmat, skb8 = pl.pallas_call(
        _tc_slab_kernel,
        grid=(grid,),
        in_specs=[
            pl.BlockSpec((16, V), lambda i: (0, 0)),
            pl.BlockSpec((_TC_BLOCK, 16), lambda i: (i, 0)),
            pl.BlockSpec((N, 8), lambda i: (0, 0)),
        ],
        out_specs=[
            pl.BlockSpec((_TC_BLOCK, V), lambda i: (i, 0)),
            pl.BlockSpec((_TC_BLOCK, V), lambda i: (i, 0)),
            pl.BlockSpec((_TC_BLOCK, V // 16), lambda i: (i, 0)),
            pl.BlockSpec((8, V), lambda i: (0, 0)),
        ],
        out_shape=[
            jax.ShapeDtypeStruct((N, V), jnp.int32),
            jax.ShapeDtypeStruct((N, V), jnp.float32),
            jax.ShapeDtypeStruct((N, V // 16), jnp.int32),
            jax.ShapeDtypeStruct((8, V), jnp.int32),
        ],
        compiler_params=pltpu.CompilerParams(
            dimension_semantics=("arbitrary",),
            vmem_limit_bytes=100 * 1024 * 1024,
        ),
    )(vox, rays, rd_full)

    skbits = skb8[0]

    mesh = plsc.VectorSubcoreMesh(core_axis_name="c", subcore_axis_name="s")
    sc = functools.partial(
        pl.kernel,
        out_type=[jax.ShapeDtypeStruct((N,), jnp.float32)] * 5,
        mesh=mesh,
        compiler_params=pltpu.CompilerParams(needs_layout_passes=False),
        scratch_types=[
            pltpu.VMEM((V,), jnp.int32),
            pltpu.VMEM((V,), jnp.int32),
            pltpu.VMEM((V,), jnp.float32),
            pltpu.VMEM((V + PAD,), jnp.int32),
            pltpu.VMEM((V + PAD,), jnp.int32),
            pltpu.VMEM((V + PAD,), jnp.int32),
            pltpu.VMEM((V + PAD,), jnp.int32),
            pltpu.VMEM((V,), jnp.float32),
            pltpu.VMEM((V,), jnp.float32),
            pltpu.VMEM((V,), jnp.float32),
            pltpu.VMEM((V,), jnp.float32),
            pltpu.VMEM((RADIX,), jnp.int32),
            pltpu.VMEM((V // 16 + 16,), jnp.int32),
            pltpu.VMEM((RPW,), jnp.float32),
            pltpu.VMEM((RPW,), jnp.float32),
            pltpu.VMEM((RPW,), jnp.float32),
            pltpu.VMEM((RPW,), jnp.float32),
            pltpu.VMEM((RPW,), jnp.float32),
        ],
    )(_sc_render_kernel)
    r, g, b, depth, alpha = sc(keymat, xmat, cntmat, skbits, colr, colg, colb,
                               density)

    rgb = jnp.stack([r, g, b], axis=1)
    return rgb, depth, alpha


# speedup vs baseline: 1.0887x; 1.0887x over previous
import functools

import jax
import jax.numpy as jnp
from jax import lax
from jax.experimental import pallas as pl
from jax.experimental.pallas import tpu as pltpu
from jax.experimental.pallas import tpu_sc as plsc

V = 8192
N = 1024
RAY_SAMPLES = 8
INF_BITS = 0x7F800000
RADIX = 2048
RMASK = RADIX - 1
NW = 32
RPW = N // NW
PAD = 16
EXIT_LOG_T = -9.2104

_TC_BLOCK = 32


def _tc_slab_kernel(vox_ref, rays_ref, rd_ref, key_ref, x_ref, skb_ref):
    vox = vox_ref[...]
    bminx, bminy, bminz = vox[0:1], vox[1:2], vox[2:3]
    bmaxx, bmaxy, bmaxz = vox[3:4], vox[4:5], vox[5:6]
    density, morton_f = vox[6:7], vox[7:8]
    px, py, pz = vox[8:9], vox[9:10], vox[10:11]

    rays = rays_ref[...]
    ox, oy, oz = rays[:, 0:1], rays[:, 1:2], rays[:, 2:3]
    ivx, ivy, ivz = rays[:, 3:4], rays[:, 4:5], rays[:, 5:6]

    t1 = (bminx - ox) * ivx
    t2 = (bmaxx - ox) * ivx
    tn = jnp.minimum(t1, t2)
    tf = jnp.maximum(t1, t2)
    t1 = (bminy - oy) * ivy
    t2 = (bmaxy - oy) * ivy
    tn = jnp.maximum(tn, jnp.minimum(t1, t2))
    tf = jnp.minimum(tf, jnp.maximum(t1, t2))
    t1 = (bminz - oz) * ivz
    t2 = (bmaxz - oz) * ivz
    tn = jnp.maximum(tn, jnp.minimum(t1, t2))
    tf = jnp.minimum(tf, jnp.maximum(t1, t2))

    valid = (tn <= tf) & (tf > 0.0)
    tnc = jnp.maximum(tn, 0.0)
    kb = lax.bitcast_convert_type(tnc, jnp.int32)
    kb = jnp.where(tnc == 0.0, 0, kb)
    kb = jnp.where(valid, kb, INF_BITS)
    key_ref[...] = kb
    x_ref[...] = density * ((tf - tnc) * (1.0 / RAY_SAMPLES))

    rd = rd_ref[...]
    mean = jnp.mean(rd[:, 0:3], axis=0, keepdims=True)
    dots = px * mean[:, 0:1] + py * mean[:, 1:2] + pz * mean[:, 2:3]
    skf = morton_f + dots * 1e-06
    bu = lax.bitcast_convert_type(skf, jnp.uint32)
    mapped = jnp.where(bu >> 31 != 0, ~bu, bu | jnp.uint32(0x80000000))
    skb = lax.bitcast_convert_type(mapped, jnp.int32)
    skb_ref[...] = jnp.broadcast_to(skb, (8, V))


def _radix_pass(c, src_k, src_i, dst_k, dst_i, hist, shift, lane, radix):
    rmask = radix - 1
    nv = (c + 15) // 16

    def clr(m, carry):
        hist[pl.ds(m * 16, 16)] = jnp.zeros((16,), jnp.int32)
        return carry
    lax.fori_loop(0, radix // 16, clr, 0)

    def hst(j, carry):
        k = src_k[pl.ds(j * 16, 16)]
        d = lax.shift_right_logical(k, shift) & rmask
        lm = (j * 16 + lane) < c
        cnt, last = plsc.scan_count(d, mask=lm)
        plsc.addupdate_scatter(hist, [d], cnt, mask=last)
        return carry
    lax.fori_loop(0, nv, hst, 0)

    def scn(m, carry):
        h = hist[pl.ds(m * 16, 16)]
        incl = plsc.cumsum(h)
        hist[pl.ds(m * 16, 16)] = incl - h + carry
        return carry + jnp.sum(h)
    lax.fori_loop(0, radix // 16, scn, jnp.int32(0))

    def prm(j, carry):
        k = src_k[pl.ds(j * 16, 16)]
        pidx = src_i[pl.ds(j * 16, 16)]
        lm = (j * 16 + lane) < c
        d = lax.shift_right_logical(k, shift) & rmask
        cnt, last = plsc.scan_count(d, mask=lm)
        base = plsc.load_gather(hist, [d])
        pos = base + cnt - 1
        plsc.store_scatter(dst_k, [pos], k, mask=lm)
        plsc.store_scatter(dst_i, [pos], pidx, mask=lm)
        plsc.addupdate_scatter(hist, [d], cnt, mask=last)
        return carry
    lax.fori_loop(0, nv, prm, 0)


def _sc_render_kernel(key_hbm, x_hbm, skb_hbm, colr_hbm, colg_hbm,
                      colb_hbm, den_hbm, r_hbm, g_hbm, b_hbm, d_hbm, a_hbm,
                      rankv, krow, xrow, KA, IA, KB, IB,
                      colr_v, colg_v, colb_v, den_v, hist, cnts,
                      obuf_r, obuf_g, obuf_b, obuf_d, obuf_a):
    wid = lax.axis_index("s") * 2 + lax.axis_index("c")
    lane = jnp.arange(16, dtype=jnp.int32)

    pltpu.sync_copy(skb_hbm, KA.at[pl.ds(0, V)])

    def fill(j, carry):
        IA[pl.ds(j * 16, 16)] = j * 16 + lane
        return carry
    lax.fori_loop(0, V // 16, fill, 0)
    cV = jnp.int32(V)
    _radix_pass(cV, KA, IA, KB, IB, hist, 0, lane, RADIX)
    _radix_pass(cV, KB, IB, KA, IA, hist, 11, lane, RADIX)
    _radix_pass(cV, KA, IA, KB, IB, hist, 22, lane, RADIX)

    def inv(j, carry):
        ov = IB[pl.ds(j * 16, 16)]
        plsc.store_scatter(rankv, [ov], j * 16 + lane)
        return carry
    lax.fori_loop(0, V // 16, inv, 0)

    pltpu.sync_copy(colr_hbm, colr_v)
    pltpu.sync_copy(colg_hbm, colg_v)
    pltpu.sync_copy(colb_hbm, colb_v)
    pltpu.sync_copy(den_hbm, den_v)

    def ray_body(rr, carry):
        ray = wid * RPW + rr
        pltpu.sync_copy(key_hbm.at[ray], krow)
        pltpu.sync_copy(x_hbm.at[ray], xrow)

        def cntb(j, carry):
            k = krow[pl.ds(j * 16, 16)]
            m = k < INF_BITS
            pc = plsc.all_reduce_population_count(m)
            plsc.store_scatter(cnts, [jnp.full((16,), j, jnp.int32)], pc,
                               mask=lane == 0)
            return carry
        lax.fori_loop(0, V // 16, cntb, 0)

        def pfx(m, carry):
            h = cnts[pl.ds(m * 16, 16)]
            incl = plsc.cumsum(h)
            cnts[pl.ds(m * 16, 16)] = incl - h + carry
            return carry + jnp.sum(h)
        c = lax.fori_loop(0, V // 256, pfx, jnp.int32(0))

        def cmp_body(j, carry):
            k = krow[pl.ds(j * 16, 16)]
            m = k < INF_BITS
            off = cnts[pl.ds(j, 16)][0]
            plsc.store_compressed(KA.at[pl.ds(off, 16)], k, mask=m)
            plsc.store_compressed(IA.at[pl.ds(off, 16)], j * 16 + lane, mask=m)
            return carry
        lax.fori_loop(0, V // 16, cmp_body, 0)

        _radix_pass(c, KA, IA, KB, IB, hist, 0, lane, 256)
        _radix_pass(c, KB, IB, KA, IA, hist, 8, lane, 256)
        _radix_pass(c, KA, IA, KB, IB, hist, 16, lane, 256)
        _radix_pass(c, KB, IB, KA, IA, hist, 24, lane, 256)

        nv = (c + 15) // 16

        def zc(j, z):
            ks = KA[pl.ds(j * 16, 16)]
            zm = (ks == 0) & ((j * 16 + lane) < c)
            return z + jnp.sum(zm.astype(jnp.int32))
        z = lax.fori_loop(0, nv, zc, jnp.int32(0))
        zv = (z + 15) // 16

        def zload(j, carry):
            kidx = IA[pl.ds(j * 16, 16)]
            rv = plsc.load_gather(rankv, [kidx])
            KB[pl.ds(j * 16, 16)] = rv
            IB[pl.ds(j * 16, 16)] = kidx
            return carry
        lax.fori_loop(0, zv, zload, 0)
        _radix_pass(z, KB, IB, KA, IA, hist, 0, lane, 128)
        _radix_pass(z, KA, IA, KB, IB, hist, 7, lane, 128)

        def zstore(j, carry):
            pos = j * 16 + lane
            zm = pos < z
            plsc.store_scatter(IA, [pos], IB[pl.ds(j * 16, 16)], mask=zm)
            plsc.store_scatter(KA, [pos], jnp.zeros((16,), jnp.int32), mask=zm)
            return carry
        lax.fori_loop(0, zv, zstore, 0)
        zero16 = jnp.zeros((16,), jnp.float32)

        def cc(st):
            jj, sumL, sumLp, aR, aG, aB, aD = st
            return (jj < nv) & (sumL >= EXIT_LOG_T)

        def cb(st):
            jj, sumL, sumLp, aR, aG, aB, aD = st
            kidx = IA[pl.ds(jj * 16, 16)]
            ks = KA[pl.ds(jj * 16, 16)]
            lm = (jj * 16 + lane) < c
            xg = plsc.load_gather(xrow, [kidx])
            xg = jnp.where(lm, xg, 0.0)
            l = -xg
            incl = plsc.cumsum(l)
            S = sumL + (incl - l)
            T = jnp.exp(S)
            e = jnp.exp(l)
            o = 1.0 - e
            proc = (T >= 1e-4) & lm
            w = jnp.where(proc, T * o, 0.0)
            tnc = plsc.bitcast(ks, jnp.float32)
            dg = plsc.load_gather(den_v, [kidx])
            mg = tnc + 4.0 * xg / dg
            cr = plsc.load_gather(colr_v, [kidx])
            cg = plsc.load_gather(colg_v, [kidx])
            cb_ = plsc.load_gather(colb_v, [kidx])
            aR = aR + w * cr
            aG = aG + w * cg
            aB = aB + w * cb_
            aD = aD + w * mg
            sumLp = sumLp + jnp.sum(jnp.where(proc, l, 0.0))
            sumL = sumL + jnp.sum(l)
            return (jj + 1, sumL, sumLp, aR, aG, aB, aD)

        st0 = (jnp.int32(0), jnp.float32(0), jnp.float32(0),
               zero16, zero16, zero16, zero16)
        _, _, sumLp, aR, aG, aB, aD = lax.while_loop(cc, cb, st0)

        rrv = jnp.full((16,), rr, jnp.int32)
        one_lane = lane == 0
        plsc.store_scatter(obuf_r, [rrv], jnp.full((16,), jnp.sum(aR)), mask=one_lane)
        plsc.store_scatter(obuf_g, [rrv], jnp.full((16,), jnp.sum(aG)), mask=one_lane)
        plsc.store_scatter(obuf_b, [rrv], jnp.full((16,), jnp.sum(aB)), mask=one_lane)
        plsc.store_scatter(obuf_d, [rrv], jnp.full((16,), jnp.sum(aD)), mask=one_lane)
        alpha_v = 1.0 - jnp.exp(jnp.full((16,), sumLp))
        plsc.store_scatter(obuf_a, [rrv], alpha_v, mask=one_lane)
        return carry

    lax.fori_loop(0, RPW, ray_body, 0)

    base = wid * RPW
    pltpu.sync_copy(obuf_r, r_hbm.at[pl.ds(base, RPW)])
    pltpu.sync_copy(obuf_g, g_hbm.at[pl.ds(base, RPW)])
    pltpu.sync_copy(obuf_b, b_hbm.at[pl.ds(base, RPW)])
    pltpu.sync_copy(obuf_d, d_hbm.at[pl.ds(base, RPW)])
    pltpu.sync_copy(obuf_a, a_hbm.at[pl.ds(base, RPW)])


def kernel(positions, sizes, densities, colors, morton_codes, ray_origins,
           ray_directions):
    half = sizes * 0.5
    bmin = positions - half[:, None]
    bmax = positions + half[:, None]
    density = jnp.exp(densities)
    vox = jnp.zeros((16, V), jnp.float32)
    vox = vox.at[0:3].set(bmin.T)
    vox = vox.at[3:6].set(bmax.T)
    vox = vox.at[6].set(density)
    vox = vox.at[7].set(morton_codes.astype(jnp.float32))
    vox = vox.at[8:11].set(positions.T)

    inv_dir = 1.0 / (ray_directions + 1e-08)
    rays = jnp.zeros((N, 16), jnp.float32)
    rays = rays.at[:, 0:3].set(ray_origins)
    rays = rays.at[:, 3:6].set(inv_dir)
    rd_full = jnp.zeros((N, 8), jnp.float32)
    rd_full = rd_full.at[:, 0:3].set(ray_directions)

    color = jax.nn.sigmoid(colors[:, :3])
    colr = jnp.asarray(color[:, 0])
    colg = jnp.asarray(color[:, 1])
    colb = jnp.asarray(color[:, 2])

    grid = N // _TC_BLOCK
    keymat, xmat, skb8 = pl.pallas_call(
        _tc_slab_kernel,
        grid=(grid,),
        in_specs=[
            pl.BlockSpec((16, V), lambda i: (0, 0)),
            pl.BlockSpec((_TC_BLOCK, 16), lambda i: (i, 0)),
            pl.BlockSpec((N, 8), lambda i: (0, 0)),
        ],
        out_specs=[
            pl.BlockSpec((_TC_BLOCK, V), lambda i: (i, 0)),
            pl.BlockSpec((_TC_BLOCK, V), lambda i: (i, 0)),
            pl.BlockSpec((8, V), lambda i: (0, 0)),
        ],
        out_shape=[
            jax.ShapeDtypeStruct((N, V), jnp.int32),
            jax.ShapeDtypeStruct((N, V), jnp.float32),
            jax.ShapeDtypeStruct((8, V), jnp.int32),
        ],
        compiler_params=pltpu.CompilerParams(
            dimension_semantics=("arbitrary",),
            vmem_limit_bytes=100 * 1024 * 1024,
        ),
    )(vox, rays, rd_full)

    skbits = skb8[0]

    mesh = plsc.VectorSubcoreMesh(core_axis_name="c", subcore_axis_name="s")
    sc = functools.partial(
        pl.kernel,
        out_type=[jax.ShapeDtypeStruct((N,), jnp.float32)] * 5,
        mesh=mesh,
        compiler_params=pltpu.CompilerParams(needs_layout_passes=False),
        scratch_types=[
            pltpu.VMEM((V,), jnp.int32),
            pltpu.VMEM((V,), jnp.int32),
            pltpu.VMEM((V,), jnp.float32),
            pltpu.VMEM((V + PAD,), jnp.int32),
            pltpu.VMEM((V + PAD,), jnp.int32),
            pltpu.VMEM((V + PAD,), jnp.int32),
            pltpu.VMEM((V + PAD,), jnp.int32),
            pltpu.VMEM((V,), jnp.float32),
            pltpu.VMEM((V,), jnp.float32),
            pltpu.VMEM((V,), jnp.float32),
            pltpu.VMEM((V,), jnp.float32),
            pltpu.VMEM((RADIX,), jnp.int32),
            pltpu.VMEM((V // 16 + 16,), jnp.int32),
            pltpu.VMEM((RPW,), jnp.float32),
            pltpu.VMEM((RPW,), jnp.float32),
            pltpu.VMEM((RPW,), jnp.float32),
            pltpu.VMEM((RPW,), jnp.float32),
            pltpu.VMEM((RPW,), jnp.float32),
        ],
    )(_sc_render_kernel)
    r, g, b, depth, alpha = sc(keymat, xmat, skbits, colr, colg, colb,
                               density)

    rgb = jnp.stack([r, g, b], axis=1)
    return rgb, depth, alpha


# speedup vs baseline: 1.7049x; 1.5660x over previous
import functools

import jax
import jax.numpy as jnp
from jax import lax
from jax.experimental import pallas as pl
from jax.experimental.pallas import tpu as pltpu
from jax.experimental.pallas import tpu_sc as plsc

V = 8192
N = 1024
RAY_SAMPLES = 8
INF_BITS = 0x7F800000
RADIX = 2048
RMASK = RADIX - 1
NW = 32
RPW = N // NW
PAD = 16
EXIT_LOG_T = -9.2104

_TC_BLOCK = 32


def _tc_slab_kernel(vox_ref, rays_ref, rd_ref, key_ref, x_ref, skb_ref):
    vox = vox_ref[...]
    bminx, bminy, bminz = vox[0:1], vox[1:2], vox[2:3]
    bmaxx, bmaxy, bmaxz = vox[3:4], vox[4:5], vox[5:6]
    density, morton_f = vox[6:7], vox[7:8]
    px, py, pz = vox[8:9], vox[9:10], vox[10:11]

    rays = rays_ref[...]
    ox, oy, oz = rays[:, 0:1], rays[:, 1:2], rays[:, 2:3]
    ivx, ivy, ivz = rays[:, 3:4], rays[:, 4:5], rays[:, 5:6]

    t1 = (bminx - ox) * ivx
    t2 = (bmaxx - ox) * ivx
    tn = jnp.minimum(t1, t2)
    tf = jnp.maximum(t1, t2)
    t1 = (bminy - oy) * ivy
    t2 = (bmaxy - oy) * ivy
    tn = jnp.maximum(tn, jnp.minimum(t1, t2))
    tf = jnp.minimum(tf, jnp.maximum(t1, t2))
    t1 = (bminz - oz) * ivz
    t2 = (bmaxz - oz) * ivz
    tn = jnp.maximum(tn, jnp.minimum(t1, t2))
    tf = jnp.minimum(tf, jnp.maximum(t1, t2))

    valid = (tn <= tf) & (tf > 0.0)
    tnc = jnp.maximum(tn, 0.0)
    kb = lax.bitcast_convert_type(tnc, jnp.int32)
    kb = jnp.where(tnc == 0.0, 0, kb)
    kb = jnp.where(valid, kb, INF_BITS)
    key_ref[...] = kb
    x_ref[...] = density * ((tf - tnc) * (1.0 / RAY_SAMPLES))

    rd = rd_ref[...]
    mean = jnp.mean(rd[:, 0:3], axis=0, keepdims=True)
    dots = px * mean[:, 0:1] + py * mean[:, 1:2] + pz * mean[:, 2:3]
    skf = morton_f + dots * 1e-06
    bu = lax.bitcast_convert_type(skf, jnp.uint32)
    mapped = jnp.where(bu >> 31 != 0, ~bu, bu | jnp.uint32(0x80000000))
    skb = lax.bitcast_convert_type(mapped, jnp.int32)
    skb_ref[...] = jnp.broadcast_to(skb, (8, V))


def _radix_pass(c, src_k, src_i, dst_k, dst_i, hist, shift, lane, radix):
    rmask = radix - 1
    nv = (c + 15) // 16

    def clr(m, carry):
        hist[pl.ds(m * 16, 16)] = jnp.zeros((16,), jnp.int32)
        return carry
    lax.fori_loop(0, radix // 16, clr, 0)

    def hst(j, carry):
        k = src_k[pl.ds(j * 16, 16)]
        d = lax.shift_right_logical(k, shift) & rmask
        lm = (j * 16 + lane) < c
        cnt, last = plsc.scan_count(d, mask=lm)
        plsc.addupdate_scatter(hist, [d], cnt, mask=last)
        return carry
    lax.fori_loop(0, nv, hst, 0)

    def scn(m, carry):
        h = hist[pl.ds(m * 16, 16)]
        incl = plsc.cumsum(h)
        hist[pl.ds(m * 16, 16)] = incl - h + carry
        return carry + jnp.sum(h)
    lax.fori_loop(0, radix // 16, scn, jnp.int32(0))

    def prm(j, carry):
        k = src_k[pl.ds(j * 16, 16)]
        pidx = src_i[pl.ds(j * 16, 16)]
        lm = (j * 16 + lane) < c
        d = lax.shift_right_logical(k, shift) & rmask
        cnt, last = plsc.scan_count(d, mask=lm)
        base = plsc.load_gather(hist, [d])
        pos = base + cnt - 1
        plsc.store_scatter(dst_k, [pos], k, mask=lm)
        plsc.store_scatter(dst_i, [pos], pidx, mask=lm)
        plsc.addupdate_scatter(hist, [d], cnt, mask=last)
        return carry
    lax.fori_loop(0, nv, prm, 0)


def _sc_render_kernel(key_hbm, x_hbm, skb_hbm, colr_hbm, colg_hbm,
                      colb_hbm, den_hbm, r_hbm, g_hbm, b_hbm, d_hbm, a_hbm,
                      rankv, krow, xrow, KA, IA, KB, IB,
                      colr_v, colg_v, colb_v, den_v, hist, cnts,
                      obuf_r, obuf_g, obuf_b, obuf_d, obuf_a):
    wid = lax.axis_index("s") * 2 + lax.axis_index("c")
    lane = jnp.arange(16, dtype=jnp.int32)

    pltpu.sync_copy(skb_hbm, KA.at[pl.ds(0, V)])

    def fill(j, carry):
        IA[pl.ds(j * 16, 16)] = j * 16 + lane
        return carry
    lax.fori_loop(0, V // 16, fill, 0)
    cV = jnp.int32(V)
    _radix_pass(cV, KA, IA, KB, IB, hist, 0, lane, RADIX)
    _radix_pass(cV, KB, IB, KA, IA, hist, 11, lane, RADIX)
    _radix_pass(cV, KA, IA, KB, IB, hist, 22, lane, RADIX)

    def inv(j, carry):
        ov = IB[pl.ds(j * 16, 16)]
        plsc.store_scatter(rankv, [ov], j * 16 + lane)
        return carry
    lax.fori_loop(0, V // 16, inv, 0)

    pltpu.sync_copy(colr_hbm, colr_v)
    pltpu.sync_copy(colg_hbm, colg_v)
    pltpu.sync_copy(colb_hbm, colb_v)
    pltpu.sync_copy(den_hbm, den_v)

    def ray_body(rr, carry):
        ray = wid * RPW + rr
        pltpu.sync_copy(key_hbm.at[ray], krow)
        pltpu.sync_copy(x_hbm.at[ray], xrow)

        def cmp_body(j, off):
            base = j * 64
            ks = [krow[pl.ds(base + u * 16, 16)] for u in range(4)]
            ms = [k < INF_BITS for k in ks]
            ss = [jnp.sum(m.astype(jnp.int32)) for m in ms]
            for u in range(4):
                plsc.store_compressed(KA.at[pl.ds(off, 16)], ks[u], mask=ms[u])
                plsc.store_compressed(IA.at[pl.ds(off, 16)],
                                      base + u * 16 + lane, mask=ms[u])
                off = off + ss[u]
            return off
        c = lax.fori_loop(0, V // 64, cmp_body, jnp.int32(0))

        _radix_pass(c, KA, IA, KB, IB, hist, 0, lane, 256)
        _radix_pass(c, KB, IB, KA, IA, hist, 8, lane, 256)
        _radix_pass(c, KA, IA, KB, IB, hist, 16, lane, 256)
        _radix_pass(c, KB, IB, KA, IA, hist, 24, lane, 256)

        nv = (c + 15) // 16

        def zc(j, z):
            ks = KA[pl.ds(j * 16, 16)]
            zm = (ks == 0) & ((j * 16 + lane) < c)
            return z + jnp.sum(zm.astype(jnp.int32))
        z = lax.fori_loop(0, nv, zc, jnp.int32(0))
        zv = (z + 15) // 16

        def zload(j, carry):
            kidx = IA[pl.ds(j * 16, 16)]
            rv = plsc.load_gather(rankv, [kidx])
            KB[pl.ds(j * 16, 16)] = rv
            IB[pl.ds(j * 16, 16)] = kidx
            return carry
        lax.fori_loop(0, zv, zload, 0)
        _radix_pass(z, KB, IB, KA, IA, hist, 0, lane, 128)
        _radix_pass(z, KA, IA, KB, IB, hist, 7, lane, 128)

        def zstore(j, carry):
            pos = j * 16 + lane
            zm = pos < z
            plsc.store_scatter(IA, [pos], IB[pl.ds(j * 16, 16)], mask=zm)
            plsc.store_scatter(KA, [pos], jnp.zeros((16,), jnp.int32), mask=zm)
            return carry
        lax.fori_loop(0, zv, zstore, 0)
        zero16 = jnp.zeros((16,), jnp.float32)

        def cc(st):
            jj, sumL, sumLp, aR, aG, aB, aD = st
            return (jj < nv) & (sumL >= EXIT_LOG_T)

        def cb(st):
            jj, sumL, sumLp, aR, aG, aB, aD = st
            kidx = IA[pl.ds(jj * 16, 16)]
            ks = KA[pl.ds(jj * 16, 16)]
            lm = (jj * 16 + lane) < c
            xg = plsc.load_gather(xrow, [kidx])
            xg = jnp.where(lm, xg, 0.0)
            l = -xg
            incl = plsc.cumsum(l)
            S = sumL + (incl - l)
            T = jnp.exp(S)
            e = jnp.exp(l)
            o = 1.0 - e
            proc = (T >= 1e-4) & lm
            w = jnp.where(proc, T * o, 0.0)
            tnc = plsc.bitcast(ks, jnp.float32)
            dg = plsc.load_gather(den_v, [kidx])
            mg = tnc + 4.0 * xg / dg
            cr = plsc.load_gather(colr_v, [kidx])
            cg = plsc.load_gather(colg_v, [kidx])
            cb_ = plsc.load_gather(colb_v, [kidx])
            aR = aR + w * cr
            aG = aG + w * cg
            aB = aB + w * cb_
            aD = aD + w * mg
            sumLp = sumLp + jnp.sum(jnp.where(proc, l, 0.0))
            sumL = sumL + jnp.sum(l)
            return (jj + 1, sumL, sumLp, aR, aG, aB, aD)

        st0 = (jnp.int32(0), jnp.float32(0), jnp.float32(0),
               zero16, zero16, zero16, zero16)
        _, _, sumLp, aR, aG, aB, aD = lax.while_loop(cc, cb, st0)

        rrv = jnp.full((16,), rr, jnp.int32)
        one_lane = lane == 0
        plsc.store_scatter(obuf_r, [rrv], jnp.full((16,), jnp.sum(aR)), mask=one_lane)
        plsc.store_scatter(obuf_g, [rrv], jnp.full((16,), jnp.sum(aG)), mask=one_lane)
        plsc.store_scatter(obuf_b, [rrv], jnp.full((16,), jnp.sum(aB)), mask=one_lane)
        plsc.store_scatter(obuf_d, [rrv], jnp.full((16,), jnp.sum(aD)), mask=one_lane)
        alpha_v = 1.0 - jnp.exp(jnp.full((16,), sumLp))
        plsc.store_scatter(obuf_a, [rrv], alpha_v, mask=one_lane)
        return carry

    lax.fori_loop(0, RPW, ray_body, 0)

    base = wid * RPW
    pltpu.sync_copy(obuf_r, r_hbm.at[pl.ds(base, RPW)])
    pltpu.sync_copy(obuf_g, g_hbm.at[pl.ds(base, RPW)])
    pltpu.sync_copy(obuf_b, b_hbm.at[pl.ds(base, RPW)])
    pltpu.sync_copy(obuf_d, d_hbm.at[pl.ds(base, RPW)])
    pltpu.sync_copy(obuf_a, a_hbm.at[pl.ds(base, RPW)])


def kernel(positions, sizes, densities, colors, morton_codes, ray_origins,
           ray_directions):
    half = sizes * 0.5
    bmin = positions - half[:, None]
    bmax = positions + half[:, None]
    density = jnp.exp(densities)
    vox = jnp.zeros((16, V), jnp.float32)
    vox = vox.at[0:3].set(bmin.T)
    vox = vox.at[3:6].set(bmax.T)
    vox = vox.at[6].set(density)
    vox = vox.at[7].set(morton_codes.astype(jnp.float32))
    vox = vox.at[8:11].set(positions.T)

    inv_dir = 1.0 / (ray_directions + 1e-08)
    rays = jnp.zeros((N, 16), jnp.float32)
    rays = rays.at[:, 0:3].set(ray_origins)
    rays = rays.at[:, 3:6].set(inv_dir)
    rd_full = jnp.zeros((N, 8), jnp.float32)
    rd_full = rd_full.at[:, 0:3].set(ray_directions)

    color = jax.nn.sigmoid(colors[:, :3])
    colr = jnp.asarray(color[:, 0])
    colg = jnp.asarray(color[:, 1])
    colb = jnp.asarray(color[:, 2])

    grid = N // _TC_BLOCK
    keymat, xmat, skb8 = pl.pallas_call(
        _tc_slab_kernel,
        grid=(grid,),
        in_specs=[
            pl.BlockSpec((16, V), lambda i: (0, 0)),
            pl.BlockSpec((_TC_BLOCK, 16), lambda i: (i, 0)),
            pl.BlockSpec((N, 8), lambda i: (0, 0)),
        ],
        out_specs=[
            pl.BlockSpec((_TC_BLOCK, V), lambda i: (i, 0)),
            pl.BlockSpec((_TC_BLOCK, V), lambda i: (i, 0)),
            pl.BlockSpec((8, V), lambda i: (0, 0)),
        ],
        out_shape=[
            jax.ShapeDtypeStruct((N, V), jnp.int32),
            jax.ShapeDtypeStruct((N, V), jnp.float32),
            jax.ShapeDtypeStruct((8, V), jnp.int32),
        ],
        compiler_params=pltpu.CompilerParams(
            dimension_semantics=("arbitrary",),
            vmem_limit_bytes=100 * 1024 * 1024,
        ),
    )(vox, rays, rd_full)

    skbits = skb8[0]

    mesh = plsc.VectorSubcoreMesh(core_axis_name="c", subcore_axis_name="s")
    sc = functools.partial(
        pl.kernel,
        out_type=[jax.ShapeDtypeStruct((N,), jnp.float32)] * 5,
        mesh=mesh,
        compiler_params=pltpu.CompilerParams(needs_layout_passes=False),
        scratch_types=[
            pltpu.VMEM((V,), jnp.int32),
            pltpu.VMEM((V,), jnp.int32),
            pltpu.VMEM((V,), jnp.float32),
            pltpu.VMEM((V + PAD,), jnp.int32),
            pltpu.VMEM((V + PAD,), jnp.int32),
            pltpu.VMEM((V + PAD,), jnp.int32),
            pltpu.VMEM((V + PAD,), jnp.int32),
            pltpu.VMEM((V,), jnp.float32),
            pltpu.VMEM((V,), jnp.float32),
            pltpu.VMEM((V,), jnp.float32),
            pltpu.VMEM((V,), jnp.float32),
            pltpu.VMEM((RADIX,), jnp.int32),
            pltpu.VMEM((V // 16 + 16,), jnp.int32),
            pltpu.VMEM((RPW,), jnp.float32),
            pltpu.VMEM((RPW,), jnp.float32),
            pltpu.VMEM((RPW,), jnp.float32),
            pltpu.VMEM((RPW,), jnp.float32),
            pltpu.VMEM((RPW,), jnp.float32),
        ],
    )(_sc_render_kernel)
    r, g, b, depth, alpha = sc(keymat, xmat, skbits, colr, colg, colb,
                               density)

    rgb = jnp.stack([r, g, b], axis=1)
    return rgb, depth, alpha


# speedup vs baseline: 1.9296x; 1.1318x over previous
import functools

import jax
import jax.numpy as jnp
from jax import lax
from jax.experimental import pallas as pl
from jax.experimental.pallas import tpu as pltpu
from jax.experimental.pallas import tpu_sc as plsc

V = 8192
N = 1024
RAY_SAMPLES = 8
INF_BITS = 0x7F800000
RADIX = 2048
RMASK = RADIX - 1
NW = 32
RPW = N // NW
PAD = 16
EXIT_LOG_T = -9.2104

_TC_BLOCK = 32


def _tc_slab_kernel(vox_ref, rays_ref, rd_ref, key_ref, x_ref, skb_ref):
    vox = vox_ref[...]
    bminx, bminy, bminz = vox[0:1], vox[1:2], vox[2:3]
    bmaxx, bmaxy, bmaxz = vox[3:4], vox[4:5], vox[5:6]
    density, morton_f = vox[6:7], vox[7:8]
    px, py, pz = vox[8:9], vox[9:10], vox[10:11]

    rays = rays_ref[...]
    ox, oy, oz = rays[:, 0:1], rays[:, 1:2], rays[:, 2:3]
    ivx, ivy, ivz = rays[:, 3:4], rays[:, 4:5], rays[:, 5:6]

    t1 = (bminx - ox) * ivx
    t2 = (bmaxx - ox) * ivx
    tn = jnp.minimum(t1, t2)
    tf = jnp.maximum(t1, t2)
    t1 = (bminy - oy) * ivy
    t2 = (bmaxy - oy) * ivy
    tn = jnp.maximum(tn, jnp.minimum(t1, t2))
    tf = jnp.minimum(tf, jnp.maximum(t1, t2))
    t1 = (bminz - oz) * ivz
    t2 = (bmaxz - oz) * ivz
    tn = jnp.maximum(tn, jnp.minimum(t1, t2))
    tf = jnp.minimum(tf, jnp.maximum(t1, t2))

    valid = (tn <= tf) & (tf > 0.0)
    tnc = jnp.maximum(tn, 0.0)
    kb = lax.bitcast_convert_type(tnc, jnp.int32)
    kb = jnp.where(tnc == 0.0, 0, kb)
    kb = jnp.where(valid, kb, INF_BITS)
    key_ref[...] = kb
    x_ref[...] = density * ((tf - tnc) * (1.0 / RAY_SAMPLES))

    rd = rd_ref[...]
    mean = jnp.mean(rd[:, 0:3], axis=0, keepdims=True)
    dots = px * mean[:, 0:1] + py * mean[:, 1:2] + pz * mean[:, 2:3]
    skf = morton_f + dots * 1e-06
    bu = lax.bitcast_convert_type(skf, jnp.uint32)
    mapped = jnp.where(bu >> 31 != 0, ~bu, bu | jnp.uint32(0x80000000))
    skb = lax.bitcast_convert_type(mapped, jnp.int32)
    skb_ref[...] = jnp.broadcast_to(skb, (8, V))


def _radix_pass(c, src_k, src_i, dst_k, dst_i, hist, shift, lane, radix,
                unroll=1):
    rmask = radix - 1
    nvu = (c + 16 * unroll - 1) // (16 * unroll)

    def clr(m, carry):
        hist[pl.ds(m * 16, 16)] = jnp.zeros((16,), jnp.int32)
        return carry
    lax.fori_loop(0, radix // 16, clr, 0)

    def hst(j, carry):
        for u in range(unroll):
            k = src_k[pl.ds(j * 16 * unroll + u * 16, 16)]
            d = lax.shift_right_logical(k, shift) & rmask
            lm = (j * 16 * unroll + u * 16 + lane) < c
            cnt, last = plsc.scan_count(d, mask=lm)
            plsc.addupdate_scatter(hist, [d], cnt, mask=last)
        return carry
    lax.fori_loop(0, nvu, hst, 0)

    def scn(m, carry):
        h = hist[pl.ds(m * 16, 16)]
        incl = plsc.cumsum(h)
        hist[pl.ds(m * 16, 16)] = incl - h + carry
        return carry + jnp.sum(h)
    lax.fori_loop(0, radix // 16, scn, jnp.int32(0))

    def prm(j, carry):
        for u in range(unroll):
            k = src_k[pl.ds(j * 16 * unroll + u * 16, 16)]
            pidx = src_i[pl.ds(j * 16 * unroll + u * 16, 16)]
            lm = (j * 16 * unroll + u * 16 + lane) < c
            d = lax.shift_right_logical(k, shift) & rmask
            cnt, last = plsc.scan_count(d, mask=lm)
            base = plsc.load_gather(hist, [d])
            pos = base + cnt - 1
            plsc.store_scatter(dst_k, [pos], k, mask=lm)
            plsc.store_scatter(dst_i, [pos], pidx, mask=lm)
            plsc.addupdate_scatter(hist, [d], cnt, mask=last)
        return carry
    lax.fori_loop(0, nvu, prm, 0)


def _sc_render_kernel(key_hbm, x_hbm, skb_hbm, colr_hbm, colg_hbm,
                      colb_hbm, den_hbm, r_hbm, g_hbm, b_hbm, d_hbm, a_hbm,
                      rankv, krow, xrow, krow2, xrow2, KA, IA, KB, IB,
                      colr_v, colg_v, colb_v, den_v, hist, cnts,
                      obuf_r, obuf_g, obuf_b, obuf_d, obuf_a, semA, semB):
    wid = lax.axis_index("s") * 2 + lax.axis_index("c")
    lane = jnp.arange(16, dtype=jnp.int32)

    pltpu.sync_copy(skb_hbm, KA.at[pl.ds(0, V)])

    def fill(j, carry):
        for u in range(4):
            IA[pl.ds(j * 64 + u * 16, 16)] = j * 64 + u * 16 + lane
        return carry
    lax.fori_loop(0, V // 64, fill, 0)
    cV = jnp.int32(V)
    _radix_pass(cV, KA, IA, KB, IB, hist, 0, lane, RADIX, unroll=4)
    _radix_pass(cV, KB, IB, KA, IA, hist, 11, lane, RADIX, unroll=4)
    _radix_pass(cV, KA, IA, KB, IB, hist, 22, lane, RADIX, unroll=4)

    def inv(j, carry):
        for u in range(4):
            ov = IB[pl.ds(j * 64 + u * 16, 16)]
            plsc.store_scatter(rankv, [ov], j * 64 + u * 16 + lane)
        return carry
    lax.fori_loop(0, V // 64, inv, 0)

    pltpu.sync_copy(colr_hbm, colr_v)
    pltpu.sync_copy(colg_hbm, colg_v)
    pltpu.sync_copy(colb_hbm, colb_v)
    pltpu.sync_copy(den_hbm, den_v)

    ray0 = wid * RPW
    pltpu.async_copy(key_hbm.at[ray0], krow, semA)
    pltpu.async_copy(x_hbm.at[ray0], xrow, semA)

    def ray_body(rr, krow, xrow, semc, krow_n, xrow_n, semn):
        ray = wid * RPW + rr
        pltpu.make_async_copy(key_hbm.at[ray], krow, semc).wait()
        pltpu.make_async_copy(x_hbm.at[ray], xrow, semc).wait()

        @pl.when(rr + 1 < RPW)
        def _():
            pltpu.async_copy(key_hbm.at[ray + 1], krow_n, semn)
            pltpu.async_copy(x_hbm.at[ray + 1], xrow_n, semn)

        def cmp_body(j, off):
            base = j * 64
            ks = [krow[pl.ds(base + u * 16, 16)] for u in range(4)]
            ms = [k < INF_BITS for k in ks]
            ss = [jnp.sum(m.astype(jnp.int32)) for m in ms]
            for u in range(4):
                plsc.store_compressed(KA.at[pl.ds(off, 16)], ks[u], mask=ms[u])
                plsc.store_compressed(IA.at[pl.ds(off, 16)],
                                      base + u * 16 + lane, mask=ms[u])
                off = off + ss[u]
            return off
        c = lax.fori_loop(0, V // 64, cmp_body, jnp.int32(0))

        _radix_pass(c, KA, IA, KB, IB, hist, 0, lane, 256)
        _radix_pass(c, KB, IB, KA, IA, hist, 8, lane, 256)
        _radix_pass(c, KA, IA, KB, IB, hist, 16, lane, 256)
        _radix_pass(c, KB, IB, KA, IA, hist, 24, lane, 128)

        nv = (c + 15) // 16

        def zc(j, z):
            ks = KA[pl.ds(j * 16, 16)]
            zm = (ks == 0) & ((j * 16 + lane) < c)
            return z + jnp.sum(zm.astype(jnp.int32))
        z = lax.fori_loop(0, nv, zc, jnp.int32(0))
        zv = (z + 15) // 16

        def zload(j, carry):
            kidx = IA[pl.ds(j * 16, 16)]
            rv = plsc.load_gather(rankv, [kidx])
            KB[pl.ds(j * 16, 16)] = rv
            IB[pl.ds(j * 16, 16)] = kidx
            return carry
        lax.fori_loop(0, zv, zload, 0)
        _radix_pass(z, KB, IB, KA, IA, hist, 0, lane, 128)
        _radix_pass(z, KA, IA, KB, IB, hist, 7, lane, 128)

        def zstore(j, carry):
            pos = j * 16 + lane
            zm = pos < z
            plsc.store_scatter(IA, [pos], IB[pl.ds(j * 16, 16)], mask=zm)
            plsc.store_scatter(KA, [pos], jnp.zeros((16,), jnp.int32), mask=zm)
            return carry
        lax.fori_loop(0, zv, zstore, 0)
        zero16 = jnp.zeros((16,), jnp.float32)

        def cc(st):
            jj, sumL, sumLp, aR, aG, aB, aD = st
            return (jj < nv) & (sumL >= EXIT_LOG_T)

        def cb(st):
            jj, sumL, sumLp, aR, aG, aB, aD = st
            kidx = IA[pl.ds(jj * 16, 16)]
            ks = KA[pl.ds(jj * 16, 16)]
            lm = (jj * 16 + lane) < c
            xg = plsc.load_gather(xrow, [kidx])
            xg = jnp.where(lm, xg, 0.0)
            l = -xg
            incl = plsc.cumsum(l)
            S = sumL + (incl - l)
            T = jnp.exp(S)
            e = jnp.exp(l)
            o = 1.0 - e
            proc = (T >= 1e-4) & lm
            w = jnp.where(proc, T * o, 0.0)
            tnc = plsc.bitcast(ks, jnp.float32)
            dg = plsc.load_gather(den_v, [kidx])
            mg = tnc + 4.0 * xg / dg
            cr = plsc.load_gather(colr_v, [kidx])
            cg = plsc.load_gather(colg_v, [kidx])
            cb_ = plsc.load_gather(colb_v, [kidx])
            aR = aR + w * cr
            aG = aG + w * cg
            aB = aB + w * cb_
            aD = aD + w * mg
            sumLp = sumLp + jnp.sum(jnp.where(proc, l, 0.0))
            sumL = sumL + jnp.sum(l)
            return (jj + 1, sumL, sumLp, aR, aG, aB, aD)

        st0 = (jnp.int32(0), jnp.float32(0), jnp.float32(0),
               zero16, zero16, zero16, zero16)
        _, _, sumLp, aR, aG, aB, aD = lax.while_loop(cc, cb, st0)

        rrv = jnp.full((16,), rr, jnp.int32)
        one_lane = lane == 0
        plsc.store_scatter(obuf_r, [rrv], jnp.full((16,), jnp.sum(aR)), mask=one_lane)
        plsc.store_scatter(obuf_g, [rrv], jnp.full((16,), jnp.sum(aG)), mask=one_lane)
        plsc.store_scatter(obuf_b, [rrv], jnp.full((16,), jnp.sum(aB)), mask=one_lane)
        plsc.store_scatter(obuf_d, [rrv], jnp.full((16,), jnp.sum(aD)), mask=one_lane)
        alpha_v = 1.0 - jnp.exp(jnp.full((16,), sumLp))
        plsc.store_scatter(obuf_a, [rrv], alpha_v, mask=one_lane)

    def pair_body(rp, carry):
        ray_body(rp * 2, krow, xrow, semA, krow2, xrow2, semB)
        ray_body(rp * 2 + 1, krow2, xrow2, semB, krow, xrow, semA)
        return carry

    lax.fori_loop(0, RPW // 2, pair_body, 0)

    base = wid * RPW
    pltpu.sync_copy(obuf_r, r_hbm.at[pl.ds(base, RPW)])
    pltpu.sync_copy(obuf_g, g_hbm.at[pl.ds(base, RPW)])
    pltpu.sync_copy(obuf_b, b_hbm.at[pl.ds(base, RPW)])
    pltpu.sync_copy(obuf_d, d_hbm.at[pl.ds(base, RPW)])
    pltpu.sync_copy(obuf_a, a_hbm.at[pl.ds(base, RPW)])


def kernel(positions, sizes, densities, colors, morton_codes, ray_origins,
           ray_directions):
    half = sizes * 0.5
    bmin = positions - half[:, None]
    bmax = positions + half[:, None]
    density = jnp.exp(densities)
    vox = jnp.zeros((16, V), jnp.float32)
    vox = vox.at[0:3].set(bmin.T)
    vox = vox.at[3:6].set(bmax.T)
    vox = vox.at[6].set(density)
    vox = vox.at[7].set(morton_codes.astype(jnp.float32))
    vox = vox.at[8:11].set(positions.T)

    inv_dir = 1.0 / (ray_directions + 1e-08)
    rays = jnp.zeros((N, 16), jnp.float32)
    rays = rays.at[:, 0:3].set(ray_origins)
    rays = rays.at[:, 3:6].set(inv_dir)
    rd_full = jnp.zeros((N, 8), jnp.float32)
    rd_full = rd_full.at[:, 0:3].set(ray_directions)

    color = jax.nn.sigmoid(colors[:, :3])
    colr = jnp.asarray(color[:, 0])
    colg = jnp.asarray(color[:, 1])
    colb = jnp.asarray(color[:, 2])

    grid = N // _TC_BLOCK
    keymat, xmat, skb8 = pl.pallas_call(
        _tc_slab_kernel,
        grid=(grid,),
        in_specs=[
            pl.BlockSpec((16, V), lambda i: (0, 0)),
            pl.BlockSpec((_TC_BLOCK, 16), lambda i: (i, 0)),
            pl.BlockSpec((N, 8), lambda i: (0, 0)),
        ],
        out_specs=[
            pl.BlockSpec((_TC_BLOCK, V), lambda i: (i, 0)),
            pl.BlockSpec((_TC_BLOCK, V), lambda i: (i, 0)),
            pl.BlockSpec((8, V), lambda i: (0, 0)),
        ],
        out_shape=[
            jax.ShapeDtypeStruct((N, V), jnp.int32),
            jax.ShapeDtypeStruct((N, V), jnp.float32),
            jax.ShapeDtypeStruct((8, V), jnp.int32),
        ],
        compiler_params=pltpu.CompilerParams(
            dimension_semantics=("arbitrary",),
            vmem_limit_bytes=100 * 1024 * 1024,
        ),
    )(vox, rays, rd_full)

    skbits = skb8[0]

    mesh = plsc.VectorSubcoreMesh(core_axis_name="c", subcore_axis_name="s")
    sc = functools.partial(
        pl.kernel,
        out_type=[jax.ShapeDtypeStruct((N,), jnp.float32)] * 5,
        mesh=mesh,
        compiler_params=pltpu.CompilerParams(needs_layout_passes=False),
        scratch_types=[
            pltpu.VMEM((V,), jnp.int32),
            pltpu.VMEM((V,), jnp.int32),
            pltpu.VMEM((V,), jnp.float32),
            pltpu.VMEM((V,), jnp.int32),
            pltpu.VMEM((V,), jnp.float32),
            pltpu.VMEM((V + PAD,), jnp.int32),
            pltpu.VMEM((V + PAD,), jnp.int32),
            pltpu.VMEM((V + PAD,), jnp.int32),
            pltpu.VMEM((V + PAD,), jnp.int32),
            pltpu.VMEM((V,), jnp.float32),
            pltpu.VMEM((V,), jnp.float32),
            pltpu.VMEM((V,), jnp.float32),
            pltpu.VMEM((V,), jnp.float32),
            pltpu.VMEM((RADIX,), jnp.int32),
            pltpu.VMEM((V // 16 + 16,), jnp.int32),
            pltpu.VMEM((RPW,), jnp.float32),
            pltpu.VMEM((RPW,), jnp.float32),
            pltpu.VMEM((RPW,), jnp.float32),
            pltpu.VMEM((RPW,), jnp.float32),
            pltpu.VMEM((RPW,), jnp.float32),
            pltpu.SemaphoreType.DMA,
            pltpu.SemaphoreType.DMA,
        ],
    )(_sc_render_kernel)
    r, g, b, depth, alpha = sc(keymat, xmat, skbits, colr, colg, colb,
                               density)

    rgb = jnp.stack([r, g, b], axis=1)
    return rgb, depth, alpha


# speedup vs baseline: 2.0060x; 1.0396x over previous
import functools

import jax
import jax.numpy as jnp
from jax import lax
from jax.experimental import pallas as pl
from jax.experimental.pallas import tpu as pltpu
from jax.experimental.pallas import tpu_sc as plsc

V = 8192
N = 1024
RAY_SAMPLES = 8
INF_BITS = 0x7F800000
RADIX = 2048
RMASK = RADIX - 1
NW = 32
RPW = N // NW
PAD = 16
EXIT_LOG_T = -9.2104

_TC_BLOCK = 32


def _tc_slab_kernel(vox_ref, rays_ref, rd_ref, key_ref, x_ref, skb_ref):
    vox = vox_ref[...]
    bminx, bminy, bminz = vox[0:1], vox[1:2], vox[2:3]
    bmaxx, bmaxy, bmaxz = vox[3:4], vox[4:5], vox[5:6]
    density, morton_f = vox[6:7], vox[7:8]
    px, py, pz = vox[8:9], vox[9:10], vox[10:11]

    rays = rays_ref[...]
    ox, oy, oz = rays[:, 0:1], rays[:, 1:2], rays[:, 2:3]
    ivx, ivy, ivz = rays[:, 3:4], rays[:, 4:5], rays[:, 5:6]

    t1 = (bminx - ox) * ivx
    t2 = (bmaxx - ox) * ivx
    tn = jnp.minimum(t1, t2)
    tf = jnp.maximum(t1, t2)
    t1 = (bminy - oy) * ivy
    t2 = (bmaxy - oy) * ivy
    tn = jnp.maximum(tn, jnp.minimum(t1, t2))
    tf = jnp.minimum(tf, jnp.maximum(t1, t2))
    t1 = (bminz - oz) * ivz
    t2 = (bmaxz - oz) * ivz
    tn = jnp.maximum(tn, jnp.minimum(t1, t2))
    tf = jnp.minimum(tf, jnp.maximum(t1, t2))

    valid = (tn <= tf) & (tf > 0.0)
    tnc = jnp.maximum(tn, 0.0)
    kb = lax.bitcast_convert_type(tnc, jnp.int32)
    kb = jnp.where(tnc == 0.0, 0, kb)
    kb = jnp.where(valid, kb, INF_BITS)
    key_ref[...] = kb
    x_ref[...] = density * ((tf - tnc) * (1.0 / RAY_SAMPLES))

    rd = rd_ref[...]
    mean = jnp.mean(rd[:, 0:3], axis=0, keepdims=True)
    dots = px * mean[:, 0:1] + py * mean[:, 1:2] + pz * mean[:, 2:3]
    skf = morton_f + dots * 1e-06
    bu = lax.bitcast_convert_type(skf, jnp.uint32)
    mapped = jnp.where(bu >> 31 != 0, ~bu, bu | jnp.uint32(0x80000000))
    skb = lax.bitcast_convert_type(mapped, jnp.int32)
    skb_ref[...] = jnp.broadcast_to(skb, (8, V))


def _radix_pass(c, src_k, src_i, dst_k, dst_i, hist, shift, lane, radix,
                unroll=1):
    rmask = radix - 1
    nvu = (c + 16 * unroll - 1) // (16 * unroll)

    def clr(m, carry):
        hist[pl.ds(m * 16, 16)] = jnp.zeros((16,), jnp.int32)
        return carry
    lax.fori_loop(0, radix // 16, clr, 0)

    def hst(j, carry):
        for u in range(unroll):
            k = src_k[pl.ds(j * 16 * unroll + u * 16, 16)]
            d = lax.shift_right_logical(k, shift) & rmask
            lm = (j * 16 * unroll + u * 16 + lane) < c
            cnt, last = plsc.scan_count(d, mask=lm)
            plsc.addupdate_scatter(hist, [d], cnt, mask=last)
        return carry
    lax.fori_loop(0, nvu, hst, 0)

    def scn(m, carry):
        h = hist[pl.ds(m * 16, 16)]
        incl = plsc.cumsum(h)
        hist[pl.ds(m * 16, 16)] = incl - h + carry
        return carry + jnp.sum(h)
    lax.fori_loop(0, radix // 16, scn, jnp.int32(0))

    def prm(j, carry):
        for u in range(unroll):
            k = src_k[pl.ds(j * 16 * unroll + u * 16, 16)]
            pidx = src_i[pl.ds(j * 16 * unroll + u * 16, 16)]
            lm = (j * 16 * unroll + u * 16 + lane) < c
            d = lax.shift_right_logical(k, shift) & rmask
            cnt, last = plsc.scan_count(d, mask=lm)
            base = plsc.load_gather(hist, [d])
            pos = base + cnt - 1
            plsc.store_scatter(dst_k, [pos], k, mask=lm)
            plsc.store_scatter(dst_i, [pos], pidx, mask=lm)
            plsc.addupdate_scatter(hist, [d], cnt, mask=last)
        return carry
    lax.fori_loop(0, nvu, prm, 0)


def _sc_render_kernel(key_hbm, x_hbm, skb_hbm, colr_hbm, colg_hbm,
                      colb_hbm, den_hbm, r_hbm, g_hbm, b_hbm, d_hbm, a_hbm,
                      rankv, krow, xrow, krow2, xrow2, KA, IA, KB, IB,
                      colr_v, colg_v, colb_v, den_v, hist, cnts,
                      obuf_r, obuf_g, obuf_b, obuf_d, obuf_a, semA, semB):
    wid = lax.axis_index("s") * 2 + lax.axis_index("c")
    lane = jnp.arange(16, dtype=jnp.int32)

    pltpu.sync_copy(skb_hbm, KA.at[pl.ds(0, V)])

    def fill(j, carry):
        for u in range(4):
            IA[pl.ds(j * 64 + u * 16, 16)] = j * 64 + u * 16 + lane
        return carry
    lax.fori_loop(0, V // 64, fill, 0)
    cV = jnp.int32(V)
    _radix_pass(cV, KA, IA, KB, IB, hist, 0, lane, RADIX, unroll=4)
    _radix_pass(cV, KB, IB, KA, IA, hist, 11, lane, RADIX, unroll=4)
    _radix_pass(cV, KA, IA, KB, IB, hist, 22, lane, RADIX, unroll=4)

    def inv(j, carry):
        for u in range(4):
            ov = IB[pl.ds(j * 64 + u * 16, 16)]
            plsc.store_scatter(rankv, [ov], j * 64 + u * 16 + lane)
        return carry
    lax.fori_loop(0, V // 64, inv, 0)

    pltpu.sync_copy(colr_hbm, colr_v)
    pltpu.sync_copy(colg_hbm, colg_v)
    pltpu.sync_copy(colb_hbm, colb_v)
    pltpu.sync_copy(den_hbm, den_v)

    ray0 = wid * RPW
    pltpu.async_copy(key_hbm.at[ray0], krow, semA)
    pltpu.async_copy(x_hbm.at[ray0], xrow, semA)

    def ray_body(rr, krow, xrow, semc, krow_n, xrow_n, semn):
        ray = wid * RPW + rr
        pltpu.make_async_copy(key_hbm.at[ray], krow, semc).wait()
        pltpu.make_async_copy(x_hbm.at[ray], xrow, semc).wait()

        @pl.when(rr + 1 < RPW)
        def _():
            pltpu.async_copy(key_hbm.at[ray + 1], krow_n, semn)
            pltpu.async_copy(x_hbm.at[ray + 1], xrow_n, semn)

        def cmp_body(j, off):
            base = j * 64
            ks = [krow[pl.ds(base + u * 16, 16)] for u in range(4)]
            ms = [k < INF_BITS for k in ks]
            ss = [plsc.all_reduce_population_count(m)[0] for m in ms]
            for u in range(4):
                plsc.store_compressed(KA.at[pl.ds(off, 16)], ks[u], mask=ms[u])
                plsc.store_compressed(IA.at[pl.ds(off, 16)],
                                      base + u * 16 + lane, mask=ms[u])
                off = off + ss[u]
            return off
        c = lax.fori_loop(0, V // 64, cmp_body, jnp.int32(0))

        _radix_pass(c, KA, IA, KB, IB, hist, 0, lane, 256)
        _radix_pass(c, KB, IB, KA, IA, hist, 8, lane, 256)
        _radix_pass(c, KA, IA, KB, IB, hist, 16, lane, 256)
        _radix_pass(c, KB, IB, KA, IA, hist, 24, lane, 128)

        nv = (c + 15) // 16

        def zc(j, z):
            ks = KA[pl.ds(j * 16, 16)]
            zm = (ks == 0) & ((j * 16 + lane) < c)
            return z + jnp.sum(zm.astype(jnp.int32))
        z = lax.fori_loop(0, nv, zc, jnp.int32(0))
        zv = (z + 15) // 16

        def zload(j, carry):
            kidx = IA[pl.ds(j * 16, 16)]
            rv = plsc.load_gather(rankv, [kidx])
            KB[pl.ds(j * 16, 16)] = rv
            IB[pl.ds(j * 16, 16)] = kidx
            return carry
        lax.fori_loop(0, zv, zload, 0)
        _radix_pass(z, KB, IB, KA, IA, hist, 0, lane, 128)
        _radix_pass(z, KA, IA, KB, IB, hist, 7, lane, 128)

        def zstore(j, carry):
            pos = j * 16 + lane
            zm = pos < z
            plsc.store_scatter(IA, [pos], IB[pl.ds(j * 16, 16)], mask=zm)
            plsc.store_scatter(KA, [pos], jnp.zeros((16,), jnp.int32), mask=zm)
            return carry
        lax.fori_loop(0, zv, zstore, 0)
        zero16 = jnp.zeros((16,), jnp.float32)

        def cc(st):
            jj, sumL, sumLp, aR, aG, aB, aD = st
            return (jj < nv) & (sumL >= EXIT_LOG_T)

        def cb(st):
            jj, sumL, sumLp, aR, aG, aB, aD = st
            kidx = IA[pl.ds(jj * 16, 16)]
            ks = KA[pl.ds(jj * 16, 16)]
            lm = (jj * 16 + lane) < c
            xg = plsc.load_gather(xrow, [kidx])
            xg = jnp.where(lm, xg, 0.0)
            l = -xg
            incl = plsc.cumsum(l)
            S = sumL + (incl - l)
            T = jnp.exp(S)
            e = jnp.exp(l)
            o = 1.0 - e
            proc = (T >= 1e-4) & lm
            w = jnp.where(proc, T * o, 0.0)
            tnc = plsc.bitcast(ks, jnp.float32)
            dg = plsc.load_gather(den_v, [kidx])
            mg = tnc + 4.0 * xg / dg
            cr = plsc.load_gather(colr_v, [kidx])
            cg = plsc.load_gather(colg_v, [kidx])
            cb_ = plsc.load_gather(colb_v, [kidx])
            aR = aR + w * cr
            aG = aG + w * cg
            aB = aB + w * cb_
            aD = aD + w * mg
            sumLp = sumLp + jnp.sum(jnp.where(proc, l, 0.0))
            sumL = sumL + jnp.sum(l)
            return (jj + 1, sumL, sumLp, aR, aG, aB, aD)

        st0 = (jnp.int32(0), jnp.float32(0), jnp.float32(0),
               zero16, zero16, zero16, zero16)
        _, _, sumLp, aR, aG, aB, aD = lax.while_loop(cc, cb, st0)

        rrv = jnp.full((16,), rr, jnp.int32)
        one_lane = lane == 0
        plsc.store_scatter(obuf_r, [rrv], jnp.full((16,), jnp.sum(aR)), mask=one_lane)
        plsc.store_scatter(obuf_g, [rrv], jnp.full((16,), jnp.sum(aG)), mask=one_lane)
        plsc.store_scatter(obuf_b, [rrv], jnp.full((16,), jnp.sum(aB)), mask=one_lane)
        plsc.store_scatter(obuf_d, [rrv], jnp.full((16,), jnp.sum(aD)), mask=one_lane)
        alpha_v = 1.0 - jnp.exp(jnp.full((16,), sumLp))
        plsc.store_scatter(obuf_a, [rrv], alpha_v, mask=one_lane)

    def pair_body(rp, carry):
        ray_body(rp * 2, krow, xrow, semA, krow2, xrow2, semB)
        ray_body(rp * 2 + 1, krow2, xrow2, semB, krow, xrow, semA)
        return carry

    lax.fori_loop(0, RPW // 2, pair_body, 0)

    base = wid * RPW
    pltpu.sync_copy(obuf_r, r_hbm.at[pl.ds(base, RPW)])
    pltpu.sync_copy(obuf_g, g_hbm.at[pl.ds(base, RPW)])
    pltpu.sync_copy(obuf_b, b_hbm.at[pl.ds(base, RPW)])
    pltpu.sync_copy(obuf_d, d_hbm.at[pl.ds(base, RPW)])
    pltpu.sync_copy(obuf_a, a_hbm.at[pl.ds(base, RPW)])


def kernel(positions, sizes, densities, colors, morton_codes, ray_origins,
           ray_directions):
    half = sizes * 0.5
    bmin = positions - half[:, None]
    bmax = positions + half[:, None]
    density = jnp.exp(densities)
    vox = jnp.zeros((16, V), jnp.float32)
    vox = vox.at[0:3].set(bmin.T)
    vox = vox.at[3:6].set(bmax.T)
    vox = vox.at[6].set(density)
    vox = vox.at[7].set(morton_codes.astype(jnp.float32))
    vox = vox.at[8:11].set(positions.T)

    inv_dir = 1.0 / (ray_directions + 1e-08)
    rays = jnp.zeros((N, 16), jnp.float32)
    rays = rays.at[:, 0:3].set(ray_origins)
    rays = rays.at[:, 3:6].set(inv_dir)
    rd_full = jnp.zeros((N, 8), jnp.float32)
    rd_full = rd_full.at[:, 0:3].set(ray_directions)

    color = jax.nn.sigmoid(colors[:, :3])
    colr = jnp.asarray(color[:, 0])
    colg = jnp.asarray(color[:, 1])
    colb = jnp.asarray(color[:, 2])

    grid = N // _TC_BLOCK
    keymat, xmat, skb8 = pl.pallas_call(
        _tc_slab_kernel,
        grid=(grid,),
        in_specs=[
            pl.BlockSpec((16, V), lambda i: (0, 0)),
            pl.BlockSpec((_TC_BLOCK, 16), lambda i: (i, 0)),
            pl.BlockSpec((N, 8), lambda i: (0, 0)),
        ],
        out_specs=[
            pl.BlockSpec((_TC_BLOCK, V), lambda i: (i, 0)),
            pl.BlockSpec((_TC_BLOCK, V), lambda i: (i, 0)),
            pl.BlockSpec((8, V), lambda i: (0, 0)),
        ],
        out_shape=[
            jax.ShapeDtypeStruct((N, V), jnp.int32),
            jax.ShapeDtypeStruct((N, V), jnp.float32),
            jax.ShapeDtypeStruct((8, V), jnp.int32),
        ],
        compiler_params=pltpu.CompilerParams(
            dimension_semantics=("arbitrary",),
            vmem_limit_bytes=100 * 1024 * 1024,
        ),
    )(vox, rays, rd_full)

    skbits = skb8[0]

    mesh = plsc.VectorSubcoreMesh(core_axis_name="c", subcore_axis_name="s")
    sc = functools.partial(
        pl.kernel,
        out_type=[jax.ShapeDtypeStruct((N,), jnp.float32)] * 5,
        mesh=mesh,
        compiler_params=pltpu.CompilerParams(needs_layout_passes=False),
        scratch_types=[
            pltpu.VMEM((V,), jnp.int32),
            pltpu.VMEM((V,), jnp.int32),
            pltpu.VMEM((V,), jnp.float32),
            pltpu.VMEM((V,), jnp.int32),
            pltpu.VMEM((V,), jnp.float32),
            pltpu.VMEM((V + PAD,), jnp.int32),
            pltpu.VMEM((V + PAD,), jnp.int32),
            pltpu.VMEM((V + PAD,), jnp.int32),
            pltpu.VMEM((V + PAD,), jnp.int32),
            pltpu.VMEM((V,), jnp.float32),
            pltpu.VMEM((V,), jnp.float32),
            pltpu.VMEM((V,), jnp.float32),
            pltpu.VMEM((V,), jnp.float32),
            pltpu.VMEM((RADIX,), jnp.int32),
            pltpu.VMEM((V // 16 + 16,), jnp.int32),
            pltpu.VMEM((RPW,), jnp.float32),
            pltpu.VMEM((RPW,), jnp.float32),
            pltpu.VMEM((RPW,), jnp.float32),
            pltpu.VMEM((RPW,), jnp.float32),
            pltpu.VMEM((RPW,), jnp.float32),
            pltpu.SemaphoreType.DMA,
            pltpu.SemaphoreType.DMA,
        ],
    )(_sc_render_kernel)
    r, g, b, depth, alpha = sc(keymat, xmat, skbits, colr, colg, colb,
                               density)

    rgb = jnp.stack([r, g, b], axis=1)
    return rgb, depth, alpha


# speedup vs baseline: 2.0533x; 1.0236x over previous
import functools

import jax
import jax.numpy as jnp
from jax import lax
from jax.experimental import pallas as pl
from jax.experimental.pallas import tpu as pltpu
from jax.experimental.pallas import tpu_sc as plsc

V = 8192
N = 1024
RAY_SAMPLES = 8
INF_BITS = 0x7F800000
RADIX = 2048
RMASK = RADIX - 1
NW = 32
RPW = N // NW
PAD = 16
EXIT_LOG_T = -9.2104

_TC_BLOCK = 32


def _tc_slab_kernel(vox_ref, rays_ref, rd_ref, key_ref, x_ref, skb_ref):
    vox = vox_ref[...]
    bminx, bminy, bminz = vox[0:1], vox[1:2], vox[2:3]
    bmaxx, bmaxy, bmaxz = vox[3:4], vox[4:5], vox[5:6]
    density, morton_f = vox[6:7], vox[7:8]
    px, py, pz = vox[8:9], vox[9:10], vox[10:11]

    rays = rays_ref[...]
    ox, oy, oz = rays[:, 0:1], rays[:, 1:2], rays[:, 2:3]
    ivx, ivy, ivz = rays[:, 3:4], rays[:, 4:5], rays[:, 5:6]

    t1 = (bminx - ox) * ivx
    t2 = (bmaxx - ox) * ivx
    tn = jnp.minimum(t1, t2)
    tf = jnp.maximum(t1, t2)
    t1 = (bminy - oy) * ivy
    t2 = (bmaxy - oy) * ivy
    tn = jnp.maximum(tn, jnp.minimum(t1, t2))
    tf = jnp.minimum(tf, jnp.maximum(t1, t2))
    t1 = (bminz - oz) * ivz
    t2 = (bmaxz - oz) * ivz
    tn = jnp.maximum(tn, jnp.minimum(t1, t2))
    tf = jnp.minimum(tf, jnp.maximum(t1, t2))

    valid = (tn <= tf) & (tf > 0.0)
    tnc = jnp.maximum(tn, 0.0)
    kb = lax.bitcast_convert_type(tnc, jnp.int32)
    kb = jnp.where(tnc == 0.0, 0, kb)
    kb = jnp.where(valid, kb, INF_BITS)
    key_ref[...] = kb
    x_ref[...] = density * ((tf - tnc) * (1.0 / RAY_SAMPLES))

    rd = rd_ref[...]
    mean = jnp.mean(rd[:, 0:3], axis=0, keepdims=True)
    dots = px * mean[:, 0:1] + py * mean[:, 1:2] + pz * mean[:, 2:3]
    skf = morton_f + dots * 1e-06
    bu = lax.bitcast_convert_type(skf, jnp.uint32)
    mapped = jnp.where(bu >> 31 != 0, ~bu, bu | jnp.uint32(0x80000000))
    skb = lax.bitcast_convert_type(mapped, jnp.int32)
    skb_ref[...] = jnp.broadcast_to(skb, (8, V))


def _radix_pass(c, src_k, src_i, dst_k, dst_i, hist, shift, lane, radix,
                unroll=1, skip_hist=False):
    rmask = radix - 1
    nvu = (c + 16 * unroll - 1) // (16 * unroll)

    if not skip_hist:
        def clr(m, carry):
            for u in range(4):
                hist[pl.ds(m * 64 + u * 16, 16)] = jnp.zeros((16,), jnp.int32)
            return carry
        lax.fori_loop(0, radix // 64, clr, 0)

        def hst(j, carry):
            for u in range(unroll):
                k = src_k[pl.ds(j * 16 * unroll + u * 16, 16)]
                d = lax.shift_right_logical(k, shift) & rmask
                lm = (j * 16 * unroll + u * 16 + lane) < c
                cnt, last = plsc.scan_count(d, mask=lm)
                plsc.addupdate_scatter(hist, [d], cnt, mask=last)
            return carry
        lax.fori_loop(0, nvu, hst, 0)

    def scn(m, carry):
        for u in range(4):
            h = hist[pl.ds(m * 64 + u * 16, 16)]
            incl = plsc.cumsum(h)
            hist[pl.ds(m * 64 + u * 16, 16)] = incl - h + carry
            carry = carry + incl[15]
        return carry
    lax.fori_loop(0, radix // 64, scn, jnp.int32(0))

    def prm(j, carry):
        for u in range(unroll):
            k = src_k[pl.ds(j * 16 * unroll + u * 16, 16)]
            pidx = src_i[pl.ds(j * 16 * unroll + u * 16, 16)]
            lm = (j * 16 * unroll + u * 16 + lane) < c
            d = lax.shift_right_logical(k, shift) & rmask
            cnt, last = plsc.scan_count(d, mask=lm)
            base = plsc.load_gather(hist, [d])
            pos = base + cnt - 1
            plsc.store_scatter(dst_k, [pos], k, mask=lm)
            plsc.store_scatter(dst_i, [pos], pidx, mask=lm)
            plsc.addupdate_scatter(hist, [d], cnt, mask=last)
        return carry
    lax.fori_loop(0, nvu, prm, 0)


def _sc_render_kernel(key_hbm, x_hbm, skb_hbm, colr_hbm, colg_hbm,
                      colb_hbm, den_hbm, r_hbm, g_hbm, b_hbm, d_hbm, a_hbm,
                      rankv, krow, xrow, krow2, xrow2, KA, IA, KB, IB,
                      colr_v, colg_v, colb_v, den_v, hist, cnts,
                      obuf_r, obuf_g, obuf_b, obuf_d, obuf_a, semA, semB):
    wid = lax.axis_index("s") * 2 + lax.axis_index("c")
    lane = jnp.arange(16, dtype=jnp.int32)

    pltpu.sync_copy(skb_hbm, KA.at[pl.ds(0, V)])

    def fill(j, carry):
        for u in range(4):
            IA[pl.ds(j * 64 + u * 16, 16)] = j * 64 + u * 16 + lane
        return carry
    lax.fori_loop(0, V // 64, fill, 0)
    cV = jnp.int32(V)
    _radix_pass(cV, KA, IA, KB, IB, hist, 0, lane, RADIX, unroll=4)
    _radix_pass(cV, KB, IB, KA, IA, hist, 11, lane, RADIX, unroll=4)
    _radix_pass(cV, KA, IA, KB, IB, hist, 22, lane, RADIX, unroll=4)

    def inv(j, carry):
        for u in range(4):
            ov = IB[pl.ds(j * 64 + u * 16, 16)]
            plsc.store_scatter(rankv, [ov], j * 64 + u * 16 + lane)
        return carry
    lax.fori_loop(0, V // 64, inv, 0)

    pltpu.sync_copy(colr_hbm, colr_v)
    pltpu.sync_copy(colg_hbm, colg_v)
    pltpu.sync_copy(colb_hbm, colb_v)
    pltpu.sync_copy(den_hbm, den_v)

    ray0 = wid * RPW
    pltpu.async_copy(key_hbm.at[ray0], krow, semA)
    pltpu.async_copy(x_hbm.at[ray0], xrow, semA)

    def ray_body(rr, krow, xrow, semc, krow_n, xrow_n, semn):
        ray = wid * RPW + rr
        pltpu.make_async_copy(key_hbm.at[ray], krow, semc).wait()
        pltpu.make_async_copy(x_hbm.at[ray], xrow, semc).wait()

        @pl.when(rr + 1 < RPW)
        def _():
            pltpu.async_copy(key_hbm.at[ray + 1], krow_n, semn)
            pltpu.async_copy(x_hbm.at[ray + 1], xrow_n, semn)

        def clr0(m, carry):
            for u in range(4):
                hist[pl.ds(m * 64 + u * 16, 16)] = jnp.zeros((16,), jnp.int32)
            return carry
        lax.fori_loop(0, 256 // 64, clr0, 0)

        def cmp_body(j, off):
            base = j * 64
            ks = [krow[pl.ds(base + u * 16, 16)] for u in range(4)]
            ms = [k < INF_BITS for k in ks]
            ss = [plsc.all_reduce_population_count(m)[0] for m in ms]
            for u in range(4):
                d = ks[u] & 255
                cnt, last = plsc.scan_count(d, mask=ms[u])
                plsc.addupdate_scatter(hist, [d], cnt, mask=last)
                plsc.store_compressed(KA.at[pl.ds(off, 16)], ks[u], mask=ms[u])
                plsc.store_compressed(IA.at[pl.ds(off, 16)],
                                      base + u * 16 + lane, mask=ms[u])
                off = off + ss[u]
            return off
        c = lax.fori_loop(0, V // 64, cmp_body, jnp.int32(0))

        _radix_pass(c, KA, IA, KB, IB, hist, 0, lane, 256, unroll=2,
                    skip_hist=True)
        _radix_pass(c, KB, IB, KA, IA, hist, 8, lane, 256, unroll=2)
        _radix_pass(c, KA, IA, KB, IB, hist, 16, lane, 256, unroll=2)
        _radix_pass(c, KB, IB, KA, IA, hist, 24, lane, 128, unroll=2)

        nv = (c + 15) // 16

        def zc(j, z):
            ks = KA[pl.ds(j * 16, 16)]
            zm = (ks == 0) & ((j * 16 + lane) < c)
            return z + jnp.sum(zm.astype(jnp.int32))
        z = lax.fori_loop(0, nv, zc, jnp.int32(0))
        zv = (z + 15) // 16

        def zload(j, carry):
            kidx = IA[pl.ds(j * 16, 16)]
            rv = plsc.load_gather(rankv, [kidx])
            KB[pl.ds(j * 16, 16)] = rv
            IB[pl.ds(j * 16, 16)] = kidx
            return carry
        lax.fori_loop(0, zv, zload, 0)
        _radix_pass(z, KB, IB, KA, IA, hist, 0, lane, 128)
        _radix_pass(z, KA, IA, KB, IB, hist, 7, lane, 128)

        def zstore(j, carry):
            pos = j * 16 + lane
            zm = pos < z
            plsc.store_scatter(IA, [pos], IB[pl.ds(j * 16, 16)], mask=zm)
            plsc.store_scatter(KA, [pos], jnp.zeros((16,), jnp.int32), mask=zm)
            return carry
        lax.fori_loop(0, zv, zstore, 0)
        zero16 = jnp.zeros((16,), jnp.float32)

        def cc(st):
            jj, sumL, sumLp, aR, aG, aB, aD = st
            return (jj < nv) & (sumL >= EXIT_LOG_T)

        def cb(st):
            jj, sumL, sumLp, aR, aG, aB, aD = st
            kidx = IA[pl.ds(jj * 16, 16)]
            ks = KA[pl.ds(jj * 16, 16)]
            lm = (jj * 16 + lane) < c
            xg = plsc.load_gather(xrow, [kidx])
            xg = jnp.where(lm, xg, 0.0)
            l = -xg
            incl = plsc.cumsum(l)
            S = sumL + (incl - l)
            T = jnp.exp(S)
            e = jnp.exp(l)
            o = 1.0 - e
            proc = (T >= 1e-4) & lm
            w = jnp.where(proc, T * o, 0.0)
            tnc = plsc.bitcast(ks, jnp.float32)
            dg = plsc.load_gather(den_v, [kidx])
            mg = tnc + 4.0 * xg / dg
            cr = plsc.load_gather(colr_v, [kidx])
            cg = plsc.load_gather(colg_v, [kidx])
            cb_ = plsc.load_gather(colb_v, [kidx])
            aR = aR + w * cr
            aG = aG + w * cg
            aB = aB + w * cb_
            aD = aD + w * mg
            sumLp = sumLp + jnp.sum(jnp.where(proc, l, 0.0))
            sumL = sumL + jnp.sum(l)
            return (jj + 1, sumL, sumLp, aR, aG, aB, aD)

        st0 = (jnp.int32(0), jnp.float32(0), jnp.float32(0),
               zero16, zero16, zero16, zero16)
        _, _, sumLp, aR, aG, aB, aD = lax.while_loop(cc, cb, st0)

        rrv = jnp.full((16,), rr, jnp.int32)
        one_lane = lane == 0
        plsc.store_scatter(obuf_r, [rrv], jnp.full((16,), jnp.sum(aR)), mask=one_lane)
        plsc.store_scatter(obuf_g, [rrv], jnp.full((16,), jnp.sum(aG)), mask=one_lane)
        plsc.store_scatter(obuf_b, [rrv], jnp.full((16,), jnp.sum(aB)), mask=one_lane)
        plsc.store_scatter(obuf_d, [rrv], jnp.full((16,), jnp.sum(aD)), mask=one_lane)
        alpha_v = 1.0 - jnp.exp(jnp.full((16,), sumLp))
        plsc.store_scatter(obuf_a, [rrv], alpha_v, mask=one_lane)

    def pair_body(rp, carry):
        ray_body(rp * 2, krow, xrow, semA, krow2, xrow2, semB)
        ray_body(rp * 2 + 1, krow2, xrow2, semB, krow, xrow, semA)
        return carry

    lax.fori_loop(0, RPW // 2, pair_body, 0)

    base = wid * RPW
    pltpu.sync_copy(obuf_r, r_hbm.at[pl.ds(base, RPW)])
    pltpu.sync_copy(obuf_g, g_hbm.at[pl.ds(base, RPW)])
    pltpu.sync_copy(obuf_b, b_hbm.at[pl.ds(base, RPW)])
    pltpu.sync_copy(obuf_d, d_hbm.at[pl.ds(base, RPW)])
    pltpu.sync_copy(obuf_a, a_hbm.at[pl.ds(base, RPW)])


def kernel(positions, sizes, densities, colors, morton_codes, ray_origins,
           ray_directions):
    half = sizes * 0.5
    bmin = positions - half[:, None]
    bmax = positions + half[:, None]
    density = jnp.exp(densities)
    vox = jnp.zeros((16, V), jnp.float32)
    vox = vox.at[0:3].set(bmin.T)
    vox = vox.at[3:6].set(bmax.T)
    vox = vox.at[6].set(density)
    vox = vox.at[7].set(morton_codes.astype(jnp.float32))
    vox = vox.at[8:11].set(positions.T)

    inv_dir = 1.0 / (ray_directions + 1e-08)
    rays = jnp.zeros((N, 16), jnp.float32)
    rays = rays.at[:, 0:3].set(ray_origins)
    rays = rays.at[:, 3:6].set(inv_dir)
    rd_full = jnp.zeros((N, 8), jnp.float32)
    rd_full = rd_full.at[:, 0:3].set(ray_directions)

    color = jax.nn.sigmoid(colors[:, :3])
    colr = jnp.asarray(color[:, 0])
    colg = jnp.asarray(color[:, 1])
    colb = jnp.asarray(color[:, 2])

    grid = N // _TC_BLOCK
    keymat, xmat, skb8 = pl.pallas_call(
        _tc_slab_kernel,
        grid=(grid,),
        in_specs=[
            pl.BlockSpec((16, V), lambda i: (0, 0)),
            pl.BlockSpec((_TC_BLOCK, 16), lambda i: (i, 0)),
            pl.BlockSpec((N, 8), lambda i: (0, 0)),
        ],
        out_specs=[
            pl.BlockSpec((_TC_BLOCK, V), lambda i: (i, 0)),
            pl.BlockSpec((_TC_BLOCK, V), lambda i: (i, 0)),
            pl.BlockSpec((8, V), lambda i: (0, 0)),
        ],
        out_shape=[
            jax.ShapeDtypeStruct((N, V), jnp.int32),
            jax.ShapeDtypeStruct((N, V), jnp.float32),
            jax.ShapeDtypeStruct((8, V), jnp.int32),
        ],
        compiler_params=pltpu.CompilerParams(
            dimension_semantics=("arbitrary",),
            vmem_limit_bytes=100 * 1024 * 1024,
        ),
    )(vox, rays, rd_full)

    skbits = skb8[0]

    mesh = plsc.VectorSubcoreMesh(core_axis_name="c", subcore_axis_name="s")
    sc = functools.partial(
        pl.kernel,
        out_type=[jax.ShapeDtypeStruct((N,), jnp.float32)] * 5,
        mesh=mesh,
        compiler_params=pltpu.CompilerParams(needs_layout_passes=False),
        scratch_types=[
            pltpu.VMEM((V,), jnp.int32),
            pltpu.VMEM((V,), jnp.int32),
            pltpu.VMEM((V,), jnp.float32),
            pltpu.VMEM((V,), jnp.int32),
            pltpu.VMEM((V,), jnp.float32),
            pltpu.VMEM((V + PAD,), jnp.int32),
            pltpu.VMEM((V + PAD,), jnp.int32),
            pltpu.VMEM((V + PAD,), jnp.int32),
            pltpu.VMEM((V + PAD,), jnp.int32),
            pltpu.VMEM((V,), jnp.float32),
            pltpu.VMEM((V,), jnp.float32),
            pltpu.VMEM((V,), jnp.float32),
            pltpu.VMEM((V,), jnp.float32),
            pltpu.VMEM((RADIX,), jnp.int32),
            pltpu.VMEM((V // 16 + 16,), jnp.int32),
            pltpu.VMEM((RPW,), jnp.float32),
            pltpu.VMEM((RPW,), jnp.float32),
            pltpu.VMEM((RPW,), jnp.float32),
            pltpu.VMEM((RPW,), jnp.float32),
            pltpu.VMEM((RPW,), jnp.float32),
            pltpu.SemaphoreType.DMA,
            pltpu.SemaphoreType.DMA,
        ],
    )(_sc_render_kernel)
    r, g, b, depth, alpha = sc(keymat, xmat, skbits, colr, colg, colb,
                               density)

    rgb = jnp.stack([r, g, b], axis=1)
    return rgb, depth, alpha
